# baseline (speedup 1.0000x reference)
import functools

import jax
import jax.numpy as jnp
from jax import lax
from jax.experimental import pallas as pl
from jax.experimental.pallas import tpu as pltpu
from jax.experimental.pallas import tpu_sc as plsc

ROWS = 128
COLS = 32768
KTOP = 512
LANES = 16
NV = COLS // LANES

_info = plsc.get_sparse_core_info()
_NC = _info.num_cores
_NS = _info.num_subcores
NW = _NC * _NS
RPW = ROWS // NW

_BIG = jnp.int32(1 << 30)


def _scalarize(v):
    return jnp.max(v) if getattr(v, "ndim", 0) else v


def _zero(hist, nb):
    z = jnp.zeros((LANES,), jnp.int32)

    @plsc.parallel_loop(0, nb // LANES, unroll=8)
    def body(j):
        hist[pl.ds(j * LANES, LANES)] = z


def _hist_scan(hist, nb, base):
    nch = nb // LANES

    def scan_body(jj, carry):
        acc, found, jf, accf = carry
        j = nch - 1 - jj
        h = hist[pl.ds(j * LANES, LANES)]
        s = jnp.sum(h)
        newtot = acc + s
        hit = jnp.logical_and(found == 0, newtot >= KTOP)
        jf = jnp.where(hit, j, jf)
        accf = jnp.where(hit, acc, accf)
        found = jnp.where(hit, jnp.int32(1), found)
        return newtot, found, jf, accf

    _, found, jf, accf = plsc.parallel_loop(
        0, nch, unroll=4,
        carry=(base, jnp.int32(0), jnp.int32(0), base))(scan_body)
    h = hist[pl.ds(jf * LANES, LANES)]
    rev = lax.rev(h, (0,))
    cs = plsc.cumsum(rev)
    cond = (accf + cs) >= KTOP
    lane = _scalarize(plsc.all_reduce_ffs(cond))
    sel = lax.iota(jnp.int32, LANES) == lane
    cnt_ge = accf + jnp.sum(jnp.where(sel, cs, 0))
    cnt_eq = jnp.sum(jnp.where(sel, rev, 0))
    b = jf * LANES + (LANES - 1) - lane
    return b, cnt_ge, cnt_eq, found


def _process_row(buf, hist, scal):
    iota = lax.iota(jnp.int32, LANES)

    _zero(hist, 1024)

    @plsc.parallel_loop(0, NV, unroll=16)
    def h1_body(j):
        v = buf[pl.ds(j * LANES, LANES)]
        bits = plsc.bitcast(v, jnp.int32)
        pos = v > 0.0
        d1 = lax.shift_right_logical(bits, 21)
        cnt, last = plsc.scan_count(d1, mask=pos)
        plsc.addupdate_scatter(hist, [d1], cnt, mask=last)

    b1, ge1, eq1, found1 = _hist_scan(hist, 1024, jnp.int32(0))
    gt1 = ge1 - eq1

    _zero(hist, 2048)

    @plsc.parallel_loop(0, NV, unroll=16)
    def h2_body(j):
        v = buf[pl.ds(j * LANES, LANES)]
        bits = plsc.bitcast(v, jnp.int32)
        pos = v > 0.0
        d1 = lax.shift_right_logical(bits, 21)
        m1 = jnp.logical_and(pos, d1 == b1)
        d2 = jnp.bitwise_and(lax.shift_right_logical(bits, 10), 0x7FF)
        cnt, last = plsc.scan_count(d2, mask=m1)
        plsc.addupdate_scatter(hist, [d2], cnt, mask=last)

    b2, ge2, eq2, _ = _hist_scan(hist, 2048, gt1)
    gt2 = ge2 - eq2

    _zero(hist, 1024)
    hi21 = jnp.bitwise_or(lax.shift_left(b1, 11), b2)

    @plsc.parallel_loop(0, NV, unroll=16)
    def h3_body(j):
        v = buf[pl.ds(j * LANES, LANES)]
        bits = plsc.bitcast(v, jnp.int32)
        pos = v > 0.0
        d12 = lax.shift_right_logical(bits, 10)
        m2 = jnp.logical_and(pos, d12 == hi21)
        d3 = jnp.bitwise_and(bits, 0x3FF)
        cnt, last = plsc.scan_count(d3, mask=m2)
        plsc.addupdate_scatter(hist, [d3], cnt, mask=last)

    b3, ge3, eq3, _ = _hist_scan(hist, 1024, gt2)

    t_bits = jnp.where(
        found1 == 1,
        jnp.bitwise_or(
            lax.shift_left(b1, 21),
            jnp.bitwise_or(lax.shift_left(b2, 10), b3)),
        jnp.int32(0))
    cnt_gt = ge3 - eq3
    straddle = jnp.logical_and(found1 == 1, ge3 > KTOP)
    m = KTOP - cnt_gt

    scal[0] = jnp.where(found1 == 1, _BIG, jnp.int32(-1))

    @pl.when(straddle)
    def _tie():
        tb = jnp.full((LANES,), t_bits, jnp.int32)

        def cond_f(carry):
            j, cnt, cut = carry
            return jnp.logical_and(j < NV, cut < 0)

        def body_f(carry):
            j, cnt, cut = carry
            v = buf[pl.ds(j * LANES, LANES)]
            bits = plsc.bitcast(v, jnp.int32)
            eq = bits == tb
            c = _scalarize(plsc.all_reduce_population_count(eq))
            target = m - cnt
            csum = plsc.cumsum(eq.astype(jnp.int32))
            hitmask = jnp.logical_and(eq, csum == target)
            lane = _scalarize(plsc.all_reduce_ffs(hitmask))
            cut = jnp.where(c >= target, j * LANES + lane, cut)
            return j + jnp.int32(1), cnt + c, cut

        _, _, cut = lax.while_loop(
            cond_f, body_f, (jnp.int32(0), jnp.int32(0), jnp.int32(-1)))
        scal[0] = cut

    cut = scal[0]

    t_f = lax.bitcast_convert_type(t_bits, jnp.float32)
    tb_f = jnp.full((LANES,), t_f, jnp.float32)
    cutv = jnp.full((LANES,), cut, jnp.int32)
    zf = jnp.zeros((LANES,), jnp.float32)

    @plsc.parallel_loop(0, NV, unroll=16)
    def out_body(j):
        v = buf[pl.ds(j * LANES, LANES)]
        gidx = iota + j * LANES
        keep = jnp.logical_or(
            v > tb_f, jnp.logical_and(v == tb_f, gidx <= cutv))
        buf[pl.ds(j * LANES, LANES)] = jnp.where(keep, v, zf)


_mesh = plsc.VectorSubcoreMesh(core_axis_name="c", subcore_axis_name="s")


@functools.partial(
    pl.kernel,
    out_type=jax.ShapeDtypeStruct((ROWS, COLS), jnp.float32),
    mesh=_mesh,
    compiler_params=pltpu.CompilerParams(needs_layout_passes=False),
    scratch_types=[
        pltpu.VMEM((COLS,), jnp.float32),
        pltpu.VMEM((COLS,), jnp.float32),
        pltpu.VMEM((2048,), jnp.int32),
        pltpu.SMEM((8,), jnp.int32),
        pltpu.SemaphoreType.DMA,
        pltpu.SemaphoreType.DMA,
        pltpu.SemaphoreType.DMA,
        pltpu.SemaphoreType.DMA,
    ],
)
def _topk_sc(x_hbm, out_hbm, buf_a, buf_b, hist, scal,
             sem_in_a, sem_in_b, sem_out_a, sem_out_b):
    wid = lax.axis_index("s") * _NC + lax.axis_index("c")
    r0 = wid * RPW

    bufs = [buf_a, buf_b]
    sem_in = [sem_in_a, sem_in_b]
    sem_out = [sem_out_a, sem_out_b]

    in_h = [None] * RPW
    out_h = [None] * RPW
    in_h[0] = pltpu.async_copy(x_hbm.at[r0], bufs[0], sem_in[0])
    for i in range(RPW):
        cur = bufs[i % 2]
        in_h[i].wait()
        if i + 1 < RPW:
            if i >= 1:
                out_h[i - 1].wait()
            in_h[i + 1] = pltpu.async_copy(
                x_hbm.at[r0 + i + 1], bufs[(i + 1) % 2], sem_in[(i + 1) % 2])
        _process_row(cur, hist, scal)
        out_h[i] = pltpu.async_copy(cur, out_hbm.at[r0 + i], sem_out[i % 2])
    out_h[RPW - 2].wait()
    out_h[RPW - 1].wait()


def kernel(x):
    return _topk_sc(x)


# speedup vs baseline: 2.3192x; 2.3192x over previous
import functools

import jax
import jax.numpy as jnp
from jax import lax
from jax.experimental import pallas as pl
from jax.experimental.pallas import tpu as pltpu
from jax.experimental.pallas import tpu_sc as plsc

ROWS = 128
COLS = 32768
KTOP = 512
LANES = 16
NV = COLS // LANES

_info = plsc.get_sparse_core_info()
_NC = _info.num_cores
_NS = _info.num_subcores
NW = _NC * _NS
RPW = ROWS // NW

_BIG = jnp.int32(1 << 30)


def _scalarize(v):
    return jnp.max(v) if getattr(v, "ndim", 0) else v


def _zero(hist, nb):
    z = jnp.zeros((LANES,), jnp.int32)

    @plsc.parallel_loop(0, nb // LANES, unroll=8)
    def body(j):
        hist[pl.ds(j * LANES, LANES)] = z


def _hist_scan(hist, nb, base):
    nch = nb // LANES

    def scan_body(jj, carry):
        acc, found, jf, accf = carry
        j = nch - 1 - jj
        h = hist[pl.ds(j * LANES, LANES)]
        s = jnp.sum(h)
        newtot = acc + s
        hit = jnp.logical_and(found == 0, newtot >= KTOP)
        jf = jnp.where(hit, j, jf)
        accf = jnp.where(hit, acc, accf)
        found = jnp.where(hit, jnp.int32(1), found)
        return newtot, found, jf, accf

    _, found, jf, accf = plsc.parallel_loop(
        0, nch, unroll=4,
        carry=(base, jnp.int32(0), jnp.int32(0), base))(scan_body)
    h = hist[pl.ds(jf * LANES, LANES)]
    rev = lax.rev(h, (0,))
    cs = plsc.cumsum(rev)
    cond = (accf + cs) >= KTOP
    lane = _scalarize(plsc.all_reduce_ffs(cond))
    sel = lax.iota(jnp.int32, LANES) == lane
    cnt_ge = accf + jnp.sum(jnp.where(sel, cs, 0))
    cnt_eq = jnp.sum(jnp.where(sel, rev, 0))
    b = jf * LANES + (LANES - 1) - lane
    return b, cnt_ge, cnt_eq, found


def _process_row(buf, hist, scal):
    iota = lax.iota(jnp.int32, LANES)

    _zero(hist, 1024)

    @plsc.parallel_loop(0, NV, unroll=4)
    def h1_body(j):
        v = buf[pl.ds(j * LANES, LANES)]
        bits = plsc.bitcast(v, jnp.int32)
        pos = v > 0.0
        d1 = lax.shift_right_logical(bits, 21)
        cnt, last = plsc.scan_count(d1, mask=pos)
        plsc.addupdate_scatter(hist, [d1], cnt, mask=last)

    b1, ge1, eq1, found1 = _hist_scan(hist, 1024, jnp.int32(0))
    gt1 = ge1 - eq1

    _zero(hist, 2048)

    @plsc.parallel_loop(0, NV, unroll=4)
    def h2_body(j):
        v = buf[pl.ds(j * LANES, LANES)]
        bits = plsc.bitcast(v, jnp.int32)
        pos = v > 0.0
        d1 = lax.shift_right_logical(bits, 21)
        m1 = jnp.logical_and(pos, d1 == b1)
        d2 = jnp.bitwise_and(lax.shift_right_logical(bits, 10), 0x7FF)
        cnt, last = plsc.scan_count(d2, mask=m1)
        plsc.addupdate_scatter(hist, [d2], cnt, mask=last)

    b2, ge2, eq2, _ = _hist_scan(hist, 2048, gt1)
    gt2 = ge2 - eq2

    _zero(hist, 1024)
    hi21 = jnp.bitwise_or(lax.shift_left(b1, 11), b2)

    @plsc.parallel_loop(0, NV, unroll=4)
    def h3_body(j):
        v = buf[pl.ds(j * LANES, LANES)]
        bits = plsc.bitcast(v, jnp.int32)
        pos = v > 0.0
        d12 = lax.shift_right_logical(bits, 10)
        m2 = jnp.logical_and(pos, d12 == hi21)
        d3 = jnp.bitwise_and(bits, 0x3FF)
        cnt, last = plsc.scan_count(d3, mask=m2)
        plsc.addupdate_scatter(hist, [d3], cnt, mask=last)

    b3, ge3, eq3, _ = _hist_scan(hist, 1024, gt2)

    t_bits = jnp.where(
        found1 == 1,
        jnp.bitwise_or(
            lax.shift_left(b1, 21),
            jnp.bitwise_or(lax.shift_left(b2, 10), b3)),
        jnp.int32(0))
    cnt_gt = ge3 - eq3
    straddle = jnp.logical_and(found1 == 1, ge3 > KTOP)
    m = KTOP - cnt_gt

    scal[0] = jnp.where(found1 == 1, _BIG, jnp.int32(-1))

    @pl.when(straddle)
    def _tie():
        tb = jnp.full((LANES,), t_bits, jnp.int32)

        def cond_f(carry):
            j, cnt, cut = carry
            return jnp.logical_and(j < NV, cut < 0)

        def body_f(carry):
            j, cnt, cut = carry
            v = buf[pl.ds(j * LANES, LANES)]
            bits = plsc.bitcast(v, jnp.int32)
            eq = bits == tb
            c = _scalarize(plsc.all_reduce_population_count(eq))
            target = m - cnt
            csum = plsc.cumsum(eq.astype(jnp.int32))
            hitmask = jnp.logical_and(eq, csum == target)
            lane = _scalarize(plsc.all_reduce_ffs(hitmask))
            cut = jnp.where(c >= target, j * LANES + lane, cut)
            return j + jnp.int32(1), cnt + c, cut

        _, _, cut = lax.while_loop(
            cond_f, body_f, (jnp.int32(0), jnp.int32(0), jnp.int32(-1)))
        scal[0] = cut

    cut = scal[0]

    t_f = lax.bitcast_convert_type(t_bits, jnp.float32)
    tb_f = jnp.full((LANES,), t_f, jnp.float32)
    cutv = jnp.full((LANES,), cut, jnp.int32)
    zf = jnp.zeros((LANES,), jnp.float32)

    @plsc.parallel_loop(0, NV, unroll=4)
    def out_body(j):
        v = buf[pl.ds(j * LANES, LANES)]
        gidx = iota + j * LANES
        keep = jnp.logical_or(
            v > tb_f, jnp.logical_and(v == tb_f, gidx <= cutv))
        buf[pl.ds(j * LANES, LANES)] = jnp.where(keep, v, zf)


_mesh = plsc.VectorSubcoreMesh(core_axis_name="c", subcore_axis_name="s")


@functools.partial(
    pl.kernel,
    out_type=jax.ShapeDtypeStruct((ROWS, COLS), jnp.float32),
    mesh=_mesh,
    compiler_params=pltpu.CompilerParams(needs_layout_passes=False),
    scratch_types=[
        pltpu.VMEM((COLS,), jnp.float32),
        pltpu.VMEM((COLS,), jnp.float32),
        pltpu.VMEM((2048,), jnp.int32),
        pltpu.SMEM((8,), jnp.int32),
        pltpu.SemaphoreType.DMA,
        pltpu.SemaphoreType.DMA,
        pltpu.SemaphoreType.DMA,
        pltpu.SemaphoreType.DMA,
    ],
)
def _topk_sc(x_hbm, out_hbm, buf_a, buf_b, hist, scal,
             sem_in_a, sem_in_b, sem_out_a, sem_out_b):
    wid = lax.axis_index("s") * _NC + lax.axis_index("c")
    r0 = wid * RPW

    bufs = [buf_a, buf_b]
    sem_in = [sem_in_a, sem_in_b]
    sem_out = [sem_out_a, sem_out_b]

    in_h = [None] * RPW
    out_h = [None] * RPW
    in_h[0] = pltpu.async_copy(x_hbm.at[r0], bufs[0], sem_in[0])
    for i in range(RPW):
        cur = bufs[i % 2]
        in_h[i].wait()
        if i + 1 < RPW:
            if i >= 1:
                out_h[i - 1].wait()
            in_h[i + 1] = pltpu.async_copy(
                x_hbm.at[r0 + i + 1], bufs[(i + 1) % 2], sem_in[(i + 1) % 2])
        _process_row(cur, hist, scal)
        out_h[i] = pltpu.async_copy(cur, out_hbm.at[r0 + i], sem_out[i % 2])
    out_h[RPW - 2].wait()
    out_h[RPW - 1].wait()


def kernel(x):
    return _topk_sc(x)


# speedup vs baseline: 2.4775x; 1.0683x over previous
import functools

import jax
import jax.numpy as jnp
from jax import lax
from jax.experimental import pallas as pl
from jax.experimental.pallas import tpu as pltpu
from jax.experimental.pallas import tpu_sc as plsc

ROWS = 128
COLS = 32768
KTOP = 512
LANES = 16
NV = COLS // LANES

_info = plsc.get_sparse_core_info()
_NC = _info.num_cores
_NS = _info.num_subcores
NW = _NC * _NS
RPW = ROWS // NW

_BIG = jnp.int32(1 << 30)


def _scalarize(v):
    return jnp.max(v) if getattr(v, "ndim", 0) else v


def _zero(hist, nb):
    z = jnp.zeros((LANES,), jnp.int32)

    @plsc.parallel_loop(0, nb // LANES, unroll=8)
    def body(j):
        hist[pl.ds(j * LANES, LANES)] = z


def _hist_scan(hist, nb, base):
    nch = nb // LANES

    z = jnp.zeros((LANES,), jnp.int32)

    def scan_body(jj, carry):
        acc, found, jf, accf, hf = carry
        j = nch - 1 - jj
        h = hist[pl.ds(j * LANES, LANES)]
        hist[pl.ds(j * LANES, LANES)] = z
        s = jnp.sum(h)
        newtot = acc + s
        hit = jnp.logical_and(found == 0, newtot >= KTOP)
        jf = jnp.where(hit, j, jf)
        accf = jnp.where(hit, acc, accf)
        hf = jnp.where(hit, h, hf)
        found = jnp.where(hit, jnp.int32(1), found)
        return newtot, found, jf, accf, hf

    _, found, jf, accf, hf = plsc.parallel_loop(
        0, nch, unroll=4,
        carry=(base, jnp.int32(0), jnp.int32(0), base, z))(scan_body)
    rev = lax.rev(hf, (0,))
    cs = plsc.cumsum(rev)
    cond = (accf + cs) >= KTOP
    lane = _scalarize(plsc.all_reduce_ffs(cond))
    sel = lax.iota(jnp.int32, LANES) == lane
    cnt_ge = accf + jnp.sum(jnp.where(sel, cs, 0))
    cnt_eq = jnp.sum(jnp.where(sel, rev, 0))
    b = jf * LANES + (LANES - 1) - lane
    return b, cnt_ge, cnt_eq, found


def _process_row(buf, hist, scal):
    iota = lax.iota(jnp.int32, LANES)


    @plsc.parallel_loop(0, NV, unroll=8)
    def h1_body(j):
        v = buf[pl.ds(j * LANES, LANES)]
        bits = plsc.bitcast(v, jnp.int32)
        pos = v > 0.0
        d1 = lax.shift_right_logical(bits, 21)
        cnt, last = plsc.scan_count(d1, mask=pos)
        plsc.addupdate_scatter(hist, [d1], cnt, mask=last)

    b1, ge1, eq1, found1 = _hist_scan(hist, 1024, jnp.int32(0))
    gt1 = ge1 - eq1


    @plsc.parallel_loop(0, NV, unroll=8)
    def h2_body(j):
        v = buf[pl.ds(j * LANES, LANES)]
        bits = plsc.bitcast(v, jnp.int32)
        pos = v > 0.0
        d1 = lax.shift_right_logical(bits, 21)
        m1 = jnp.logical_and(pos, d1 == b1)
        d2 = jnp.bitwise_and(lax.shift_right_logical(bits, 10), 0x7FF)
        cnt, last = plsc.scan_count(d2, mask=m1)
        plsc.addupdate_scatter(hist, [d2], cnt, mask=last)

    b2, ge2, eq2, _ = _hist_scan(hist, 2048, gt1)
    gt2 = ge2 - eq2

    hi21 = jnp.bitwise_or(lax.shift_left(b1, 11), b2)

    @plsc.parallel_loop(0, NV, unroll=8)
    def h3_body(j):
        v = buf[pl.ds(j * LANES, LANES)]
        bits = plsc.bitcast(v, jnp.int32)
        pos = v > 0.0
        d12 = lax.shift_right_logical(bits, 10)
        m2 = jnp.logical_and(pos, d12 == hi21)
        d3 = jnp.bitwise_and(bits, 0x3FF)
        cnt, last = plsc.scan_count(d3, mask=m2)
        plsc.addupdate_scatter(hist, [d3], cnt, mask=last)

    b3, ge3, eq3, _ = _hist_scan(hist, 1024, gt2)

    t_bits = jnp.where(
        found1 == 1,
        jnp.bitwise_or(
            lax.shift_left(b1, 21),
            jnp.bitwise_or(lax.shift_left(b2, 10), b3)),
        jnp.int32(0))
    cnt_gt = ge3 - eq3
    straddle = jnp.logical_and(found1 == 1, ge3 > KTOP)
    m = KTOP - cnt_gt

    scal[0] = jnp.where(found1 == 1, _BIG, jnp.int32(-1))

    @pl.when(straddle)
    def _tie():
        tb = jnp.full((LANES,), t_bits, jnp.int32)

        def cond_f(carry):
            j, cnt, cut = carry
            return jnp.logical_and(j < NV, cut < 0)

        def body_f(carry):
            j, cnt, cut = carry
            v = buf[pl.ds(j * LANES, LANES)]
            bits = plsc.bitcast(v, jnp.int32)
            eq = bits == tb
            c = _scalarize(plsc.all_reduce_population_count(eq))
            target = m - cnt
            csum = plsc.cumsum(eq.astype(jnp.int32))
            hitmask = jnp.logical_and(eq, csum == target)
            lane = _scalarize(plsc.all_reduce_ffs(hitmask))
            cut = jnp.where(c >= target, j * LANES + lane, cut)
            return j + jnp.int32(1), cnt + c, cut

        _, _, cut = lax.while_loop(
            cond_f, body_f, (jnp.int32(0), jnp.int32(0), jnp.int32(-1)))
        scal[0] = cut

    cut = scal[0]

    t_f = lax.bitcast_convert_type(t_bits, jnp.float32)
    tb_f = jnp.full((LANES,), t_f, jnp.float32)
    cutv = jnp.full((LANES,), cut, jnp.int32)
    zf = jnp.zeros((LANES,), jnp.float32)

    @plsc.parallel_loop(0, NV, unroll=8)
    def out_body(j):
        v = buf[pl.ds(j * LANES, LANES)]
        gidx = iota + j * LANES
        keep = jnp.logical_or(
            v > tb_f, jnp.logical_and(v == tb_f, gidx <= cutv))
        buf[pl.ds(j * LANES, LANES)] = jnp.where(keep, v, zf)


_mesh = plsc.VectorSubcoreMesh(core_axis_name="c", subcore_axis_name="s")


@functools.partial(
    pl.kernel,
    out_type=jax.ShapeDtypeStruct((ROWS, COLS), jnp.float32),
    mesh=_mesh,
    compiler_params=pltpu.CompilerParams(needs_layout_passes=False),
    scratch_types=[
        pltpu.VMEM((COLS,), jnp.float32),
        pltpu.VMEM((COLS,), jnp.float32),
        pltpu.VMEM((2048,), jnp.int32),
        pltpu.SMEM((8,), jnp.int32),
        pltpu.SemaphoreType.DMA,
        pltpu.SemaphoreType.DMA,
        pltpu.SemaphoreType.DMA,
        pltpu.SemaphoreType.DMA,
    ],
)
def _topk_sc(x_hbm, out_hbm, buf_a, buf_b, hist, scal,
             sem_in_a, sem_in_b, sem_out_a, sem_out_b):
    wid = lax.axis_index("s") * _NC + lax.axis_index("c")
    r0 = wid * RPW
    _zero(hist, 2048)

    bufs = [buf_a, buf_b]
    sem_in = [sem_in_a, sem_in_b]
    sem_out = [sem_out_a, sem_out_b]

    in_h = [None] * RPW
    out_h = [None] * RPW
    in_h[0] = pltpu.async_copy(x_hbm.at[r0], bufs[0], sem_in[0])
    for i in range(RPW):
        cur = bufs[i % 2]
        in_h[i].wait()
        if i + 1 < RPW:
            if i >= 1:
                out_h[i - 1].wait()
            in_h[i + 1] = pltpu.async_copy(
                x_hbm.at[r0 + i + 1], bufs[(i + 1) % 2], sem_in[(i + 1) % 2])
        _process_row(cur, hist, scal)
        out_h[i] = pltpu.async_copy(cur, out_hbm.at[r0 + i], sem_out[i % 2])
    out_h[RPW - 2].wait()
    out_h[RPW - 1].wait()


def kernel(x):
    return _topk_sc(x)


# speedup vs baseline: 2.4805x; 1.0012x over previous
import functools

import jax
import jax.numpy as jnp
from jax import lax
from jax.experimental import pallas as pl
from jax.experimental.pallas import tpu as pltpu
from jax.experimental.pallas import tpu_sc as plsc

ROWS = 128
COLS = 32768
KTOP = 512
LANES = 16
NV = COLS // LANES

_info = plsc.get_sparse_core_info()
_NC = _info.num_cores
_NS = _info.num_subcores
NW = _NC * _NS
RPW = ROWS // NW

_BIG = jnp.int32(1 << 30)


def _scalarize(v):
    return jnp.max(v) if getattr(v, "ndim", 0) else v


def _zero(hist, nb):
    z = jnp.zeros((LANES,), jnp.int32)

    @plsc.parallel_loop(0, nb // LANES, unroll=8)
    def body(j):
        hist[pl.ds(j * LANES, LANES)] = z


def _hist_scan(hist, nb, base):
    nch = nb // LANES

    z = jnp.zeros((LANES,), jnp.int32)

    def scan_body(jj, carry):
        acc, found, jf, accf = carry
        j = nch - 1 - jj
        h = hist[pl.ds(j * LANES, LANES)]
        s = jnp.sum(h)
        newtot = acc + s
        hit = jnp.logical_and(found == 0, newtot >= KTOP)
        hist[pl.ds(j * LANES, LANES)] = jnp.where(hit, h, z)
        jf = jnp.where(hit, j, jf)
        accf = jnp.where(hit, acc, accf)
        found = jnp.where(hit, jnp.int32(1), found)
        return newtot, found, jf, accf

    _, found, jf, accf = plsc.parallel_loop(
        0, nch, unroll=4,
        carry=(base, jnp.int32(0), jnp.int32(0), base))(scan_body)
    h = hist[pl.ds(jf * LANES, LANES)]
    hist[pl.ds(jf * LANES, LANES)] = z
    rev = lax.rev(h, (0,))
    cs = plsc.cumsum(rev)
    cond = (accf + cs) >= KTOP
    lane = _scalarize(plsc.all_reduce_ffs(cond))
    sel = lax.iota(jnp.int32, LANES) == lane
    cnt_ge = accf + jnp.sum(jnp.where(sel, cs, 0))
    cnt_eq = jnp.sum(jnp.where(sel, rev, 0))
    b = jf * LANES + (LANES - 1) - lane
    return b, cnt_ge, cnt_eq, found


def _process_row(buf, hist, scal):
    iota = lax.iota(jnp.int32, LANES)


    @plsc.parallel_loop(0, NV, unroll=8)
    def h1_body(j):
        v = buf[pl.ds(j * LANES, LANES)]
        bits = plsc.bitcast(v, jnp.int32)
        pos = v > 0.0
        d1 = lax.shift_right_logical(bits, 21)
        cnt, last = plsc.scan_count(d1, mask=pos)
        plsc.addupdate_scatter(hist, [d1], cnt, mask=last)

    b1, ge1, eq1, found1 = _hist_scan(hist, 1024, jnp.int32(0))
    gt1 = ge1 - eq1


    @plsc.parallel_loop(0, NV, unroll=8)
    def h2_body(j):
        v = buf[pl.ds(j * LANES, LANES)]
        bits = plsc.bitcast(v, jnp.int32)
        pos = v > 0.0
        d1 = lax.shift_right_logical(bits, 21)
        m1 = jnp.logical_and(pos, d1 == b1)
        d2 = jnp.bitwise_and(lax.shift_right_logical(bits, 10), 0x7FF)
        cnt, last = plsc.scan_count(d2, mask=m1)
        plsc.addupdate_scatter(hist, [d2], cnt, mask=last)

    b2, ge2, eq2, _ = _hist_scan(hist, 2048, gt1)
    gt2 = ge2 - eq2

    hi21 = jnp.bitwise_or(lax.shift_left(b1, 11), b2)

    @plsc.parallel_loop(0, NV, unroll=8)
    def h3_body(j):
        v = buf[pl.ds(j * LANES, LANES)]
        bits = plsc.bitcast(v, jnp.int32)
        pos = v > 0.0
        d12 = lax.shift_right_logical(bits, 10)
        m2 = jnp.logical_and(pos, d12 == hi21)
        d3 = jnp.bitwise_and(bits, 0x3FF)
        cnt, last = plsc.scan_count(d3, mask=m2)
        plsc.addupdate_scatter(hist, [d3], cnt, mask=last)

    b3, ge3, eq3, _ = _hist_scan(hist, 1024, gt2)

    t_bits = jnp.where(
        found1 == 1,
        jnp.bitwise_or(
            lax.shift_left(b1, 21),
            jnp.bitwise_or(lax.shift_left(b2, 10), b3)),
        jnp.int32(0))
    cnt_gt = ge3 - eq3
    straddle = jnp.logical_and(found1 == 1, ge3 > KTOP)
    m = KTOP - cnt_gt

    scal[0] = jnp.where(found1 == 1, _BIG, jnp.int32(-1))

    @pl.when(straddle)
    def _tie():
        tb = jnp.full((LANES,), t_bits, jnp.int32)

        def cond_f(carry):
            j, cnt, cut = carry
            return jnp.logical_and(j < NV, cut < 0)

        def body_f(carry):
            j, cnt, cut = carry
            v = buf[pl.ds(j * LANES, LANES)]
            bits = plsc.bitcast(v, jnp.int32)
            eq = bits == tb
            c = _scalarize(plsc.all_reduce_population_count(eq))
            target = m - cnt
            csum = plsc.cumsum(eq.astype(jnp.int32))
            hitmask = jnp.logical_and(eq, csum == target)
            lane = _scalarize(plsc.all_reduce_ffs(hitmask))
            cut = jnp.where(c >= target, j * LANES + lane, cut)
            return j + jnp.int32(1), cnt + c, cut

        _, _, cut = lax.while_loop(
            cond_f, body_f, (jnp.int32(0), jnp.int32(0), jnp.int32(-1)))
        scal[0] = cut

    cut = scal[0]

    t_f = lax.bitcast_convert_type(t_bits, jnp.float32)
    tb_f = jnp.full((LANES,), t_f, jnp.float32)
    cutv = jnp.full((LANES,), cut, jnp.int32)
    zf = jnp.zeros((LANES,), jnp.float32)

    @plsc.parallel_loop(0, NV, unroll=8)
    def out_body(j):
        v = buf[pl.ds(j * LANES, LANES)]
        gidx = iota + j * LANES
        keep = jnp.logical_or(
            v > tb_f, jnp.logical_and(v == tb_f, gidx <= cutv))
        buf[pl.ds(j * LANES, LANES)] = jnp.where(keep, v, zf)


_mesh = plsc.VectorSubcoreMesh(core_axis_name="c", subcore_axis_name="s")


@functools.partial(
    pl.kernel,
    out_type=jax.ShapeDtypeStruct((ROWS, COLS), jnp.float32),
    mesh=_mesh,
    compiler_params=pltpu.CompilerParams(needs_layout_passes=False),
    scratch_types=[
        pltpu.VMEM((COLS,), jnp.float32),
        pltpu.VMEM((COLS,), jnp.float32),
        pltpu.VMEM((2048,), jnp.int32),
        pltpu.SMEM((8,), jnp.int32),
        pltpu.SemaphoreType.DMA,
        pltpu.SemaphoreType.DMA,
        pltpu.SemaphoreType.DMA,
        pltpu.SemaphoreType.DMA,
    ],
)
def _topk_sc(x_hbm, out_hbm, buf_a, buf_b, hist, scal,
             sem_in_a, sem_in_b, sem_out_a, sem_out_b):
    wid = lax.axis_index("s") * _NC + lax.axis_index("c")
    r0 = wid * RPW
    _zero(hist, 2048)

    bufs = [buf_a, buf_b]
    sem_in = [sem_in_a, sem_in_b]
    sem_out = [sem_out_a, sem_out_b]

    in_h = [None] * RPW
    out_h = [None] * RPW
    in_h[0] = pltpu.async_copy(x_hbm.at[r0], bufs[0], sem_in[0])
    for i in range(RPW):
        cur = bufs[i % 2]
        in_h[i].wait()
        if i + 1 < RPW:
            if i >= 1:
                out_h[i - 1].wait()
            in_h[i + 1] = pltpu.async_copy(
                x_hbm.at[r0 + i + 1], bufs[(i + 1) % 2], sem_in[(i + 1) % 2])
        _process_row(cur, hist, scal)
        out_h[i] = pltpu.async_copy(cur, out_hbm.at[r0 + i], sem_out[i % 2])
    out_h[RPW - 2].wait()
    out_h[RPW - 1].wait()


def kernel(x):
    return _topk_sc(x)
